# baseline (speedup 1.0000x reference)
import functools
import math

import jax
import jax.numpy as jnp
from jax import lax
from jax.experimental import pallas as pl
from jax.experimental.pallas import tpu as pltpu
from jax.experimental.pallas import tpu_sc as plsc

_EMB = 64
_B = 4096
_L = 200
_N = _B * _L
_NC = 2
_NS = 16
_NW = _NC * _NS
_LANES = 16
_IW = 128
_NROW = _N // _IW
_PER_W = _NROW // _NW
_NPAIR = _PER_W // 2
_SCALE = math.sqrt(_EMB)
_PROWS = 500000


def _build():
    mesh = plsc.VectorSubcoreMesh(core_axis_name="c", subcore_axis_name="s")

    @functools.partial(
        pl.kernel,
        mesh=mesh,
        out_type=jax.ShapeDtypeStruct((_NROW, _IW, _EMB), jnp.float32),
        scratch_types=[
            pltpu.VMEM((2, _IW), jnp.int32),
            pltpu.VMEM((2, _IW), jnp.int32),
            pltpu.VMEM((2, _IW, 2 * _EMB), jnp.float32),
            pltpu.VMEM((2, _IW, _EMB), jnp.float32),
            pltpu.SemaphoreType.DMA,
            pltpu.SemaphoreType.DMA,
            pltpu.SemaphoreType.DMA,
            pltpu.SemaphoreType.DMA,
        ],
    )
    def emb(tok_hbm, tab_hbm, out_hbm, tok_v, idx_v, rows_v, outb_v,
            g0, g1, s0, s1):
        wid = lax.axis_index("s") * _NC + lax.axis_index("c")
        base = pl.multiple_of(wid * _PER_W, _PER_W)
        gsems = (g0, g1)
        ssems = (s0, s1)

        def prep_idx(ci, buf):
            off = base + ci
            pltpu.sync_copy(tok_hbm.at[pl.ds(off, 1)],
                            tok_v.at[pl.ds(buf, 1)])
            for j in range(_IW // _LANES):
                sl = pl.ds(j * _LANES, _LANES)
                idx_v[buf, sl] = lax.shift_right_logical(tok_v[buf, sl], 1)

        def fire_g(buf):
            pltpu.async_copy(
                tab_hbm.at[idx_v.at[buf]],
                rows_v.at[buf],
                gsems[buf],
            )

        def wait_g(buf):
            pltpu.make_async_copy(
                tab_hbm.at[idx_v.at[buf]],
                rows_v.at[buf],
                gsems[buf],
            ).wait()

        def fire_store(buf, off):
            pltpu.async_copy(outb_v.at[pl.ds(buf, 1)],
                             out_hbm.at[pl.ds(off, 1)],
                             ssems[buf])

        def wait_store(buf):
            pltpu.make_async_copy(outb_v.at[pl.ds(buf, 1)],
                                  out_hbm.at[pl.ds(base, 1)],
                                  ssems[buf]).wait()

        def select_scale(buf):
            def row_body(r, c):
                tv = tok_v[buf, pl.ds(r, 1)]
                h = (tv[0] & 1) * _EMB
                for j in range(_EMB // _LANES):
                    src = pl.ds(h + j * _LANES, _LANES)
                    dst = pl.ds(j * _LANES, _LANES)
                    outb_v[buf, r, dst] = rows_v[buf, r, src] * _SCALE
                return c

            lax.fori_loop(0, _IW, row_body, 0, unroll=4)

        prep_idx(0, 0)
        fire_g(0)

        def pair_body(k, carry):
            i0 = 2 * k
            off0 = base + i0
            off1 = off0 + 1

            @pl.when(k > 0)
            def _():
                wait_store(1)

            prep_idx(i0 + 1, 1)
            fire_g(1)

            wait_g(0)
            select_scale(0)
            fire_store(0, off0)

            wait_g(1)

            @pl.when(k < _NPAIR - 1)
            def _():
                wait_store(0)
                prep_idx(i0 + 2, 0)
                fire_g(0)

            select_scale(1)
            fire_store(1, off1)
            return carry

        lax.fori_loop(0, _NPAIR, pair_body, 0)
        wait_store(0)
        wait_store(1)

    return emb


_emb = _build()


@jax.jit
def kernel(tokens, table):
    tok = tokens.astype(jnp.int32).reshape(_NROW, _IW)
    tab = table.reshape(_PROWS, 2 * _EMB)
    out = _emb(tok, tab)
    return out.reshape(_B, _L, _EMB)


# speedup vs baseline: 1.4010x; 1.4010x over previous
import functools
import math

import jax
import jax.numpy as jnp
from jax import lax
from jax.experimental import pallas as pl
from jax.experimental.pallas import tpu as pltpu
from jax.experimental.pallas import tpu_sc as plsc

_EMB = 64
_B = 4096
_L = 200
_NC = 2
_NS = 16
_NW = _NC * _NS
_LANES = 16
_PER_W = _B // _NW
_NPAIR = _PER_W // 2
_IW = _L // 2
_SCALE = math.sqrt(_EMB)
_VOCAB = 1000000


def _build():
    mesh = plsc.VectorSubcoreMesh(core_axis_name="c", subcore_axis_name="s")

    @functools.partial(
        pl.kernel,
        mesh=mesh,
        out_type=jax.ShapeDtypeStruct((_B, _L, _EMB), jnp.float32),
        scratch_types=[
            pltpu.VMEM((2, 2, _IW), jnp.int32),
            pltpu.VMEM((2, _L, 2 * _EMB), jnp.float32),
            pltpu.VMEM((2, _L, _EMB), jnp.float32),
            pltpu.SemaphoreType.DMA,
            pltpu.SemaphoreType.DMA,
            pltpu.SemaphoreType.DMA,
            pltpu.SemaphoreType.DMA,
        ],
    )
    def emb(tok_hbm, tab_hbm, out_hbm, idx_v, rows_v, outb_v, g0, g1, s0, s1):
        wid = lax.axis_index("s") * _NC + lax.axis_index("c")
        base = pl.multiple_of(wid * _PER_W, _PER_W)
        gsems = (g0, g1)
        ssems = (s0, s1)

        def prep_idx(ci, buf):
            off = base + ci
            pltpu.sync_copy(tok_hbm.at[pl.ds(off, 1)],
                            idx_v.at[pl.ds(buf, 1)])

        def fire_g(buf):
            for g in range(2):
                pltpu.async_copy(
                    tab_hbm.at[idx_v.at[buf, g]],
                    rows_v.at[buf, pl.ds(g * _IW, _IW)],
                    gsems[buf],
                )

        def wait_g(buf):
            for g in range(2):
                pltpu.make_async_copy(
                    tab_hbm.at[idx_v.at[buf, g]],
                    rows_v.at[buf, pl.ds(g * _IW, _IW)],
                    gsems[buf],
                ).wait()

        def fire_store(buf, off):
            pltpu.async_copy(outb_v.at[pl.ds(buf, 1)],
                             out_hbm.at[pl.ds(off, 1)],
                             ssems[buf])

        def wait_store(buf):
            pltpu.make_async_copy(outb_v.at[pl.ds(buf, 1)],
                                  out_hbm.at[pl.ds(base, 1)],
                                  ssems[buf]).wait()

        def scale(buf):
            def row_body(r, c):
                for j in range(_EMB // _LANES):
                    sl = pl.ds(j * _LANES, _LANES)
                    outb_v[buf, r, sl] = rows_v[buf, r, sl] * _SCALE
                return c

            lax.fori_loop(0, _L, row_body, 0, unroll=8)

        prep_idx(0, 0)
        fire_g(0)

        def pair_body(k, carry):
            i0 = 2 * k
            off0 = base + i0
            off1 = off0 + 1

            @pl.when(k > 0)
            def _():
                wait_store(1)

            prep_idx(i0 + 1, 1)
            fire_g(1)

            wait_g(0)
            scale(0)
            fire_store(0, off0)

            wait_g(1)

            @pl.when(k < _NPAIR - 1)
            def _():
                wait_store(0)
                prep_idx(i0 + 2, 0)
                fire_g(0)

            scale(1)
            fire_store(1, off1)
            return carry

        lax.fori_loop(0, _NPAIR, pair_body, 0)
        wait_store(0)
        wait_store(1)

    return emb


_emb = _build()


@jax.jit
def kernel(tokens, table):
    tok = tokens.astype(jnp.int32).reshape(_B, 2, _IW)
    tab = jnp.pad(table, ((0, 0), (0, _EMB)))
    return _emb(tok, tab)
